# baseline (speedup 1.0000x reference)
import functools
import jax
import jax.numpy as jnp
from jax import lax
from jax.experimental import pallas as pl
from jax.experimental.pallas import tpu as pltpu, tpu_sc as plsc

B, S, D = 4, 8192, 1024
NW = 32
SEQ_PER_W = S // NW
R = 8
GROUPS = SEQ_PER_W // R
LANES = D // 16
NS = 3
PF = 1

_mesh = plsc.VectorSubcoreMesh(core_axis_name="c", subcore_axis_name="s",
                               num_cores=2, num_subcores=16)

_xbuf = pltpu.VMEM((B, R, D), jnp.float32)
_tbuf = pltpu.VMEM((R, D), jnp.float32)


@functools.partial(
    pl.kernel,
    out_type=jax.ShapeDtypeStruct((B, S, D), jnp.float32),
    mesh=_mesh,
    scratch_types=[
        _xbuf, _xbuf, _xbuf,
        _tbuf, _tbuf, _tbuf,
        pltpu.SemaphoreType.DMA,
        pltpu.SemaphoreType.DMA,
        pltpu.SemaphoreType.DMA,
        pltpu.SemaphoreType.DMA,
        pltpu.SemaphoreType.DMA,
        pltpu.SemaphoreType.DMA,
        pltpu.SemaphoreType.DMA,
        pltpu.SemaphoreType.DMA,
        pltpu.SemaphoreType.DMA,
    ],
    compiler_params=pltpu.CompilerParams(use_tc_tiling_on_sc=True),
)
def _sc_add(x_hbm, t_hbm, o_hbm,
            x0, x1, x2, t0, t1, t2,
            lx0, lx1, lx2, lt0, lt1, lt2, ss0, ss1, ss2):
    wid = lax.axis_index("s") * 2 + lax.axis_index("c")
    base = wid * SEQ_PER_W
    xbufs = (x0, x1, x2)
    tbufs = (t0, t1, t2)
    xsems = (lx0, lx1, lx2)
    tsems = (lt0, lt1, lt2)
    ssems = (ss0, ss1, ss2)

    def start_loads(g):
        s = g % NS
        xd = pltpu.async_copy(x_hbm.at[:, pl.ds(base + g * R, R)],
                              xbufs[s], xsems[s])
        td = pltpu.async_copy(t_hbm.at[pl.ds(base + g * R, R)],
                              tbufs[s], tsems[s])
        return xd, td

    loads = [start_loads(0), None, None]
    stores = [None, None, None]

    for g in range(GROUPS):
        s = g % NS
        ng = g + PF
        if ng < GROUPS:
            if stores[ng % NS] is not None:
                stores[ng % NS].wait()
            loads[ng % NS] = start_loads(ng)
        xd, td = loads[s]
        xd.wait()
        td.wait()
        xb = xbufs[s]
        tb = tbufs[s]

        @plsc.parallel_loop(0, LANES, unroll=1)
        def _(i):
            sl = pl.ds(i * 16, 16)
            for r in range(R):
                t = tb[r, sl]
                for b in range(B):
                    plsc.addupdate(xb.at[b, r, sl], t)

        stores[s] = pltpu.async_copy(xb, o_hbm.at[:, pl.ds(base + g * R, R)],
                                     ssems[s])

    for st in stores:
        if st is not None:
            st.wait()


def kernel(x, table):
    return _sc_add(x, table)


# speedup vs baseline: 1.0017x; 1.0017x over previous
import functools
import jax
import jax.numpy as jnp
from jax import lax
from jax.experimental import pallas as pl
from jax.experimental.pallas import tpu as pltpu, tpu_sc as plsc

B, S, D = 4, 8192, 1024
NW = 32
SEQ_PER_W = S // NW
R = 8
GROUPS = SEQ_PER_W // R
LANES = D // 16
NS = 3
PF = 1

_mesh = plsc.VectorSubcoreMesh(core_axis_name="c", subcore_axis_name="s",
                               num_cores=2, num_subcores=16)

_xbuf = pltpu.VMEM((B, R, D), jnp.float32)
_tbuf = pltpu.VMEM((R, D), jnp.float32)


@functools.partial(
    pl.kernel,
    out_type=jax.ShapeDtypeStruct((B, S, D), jnp.float32),
    mesh=_mesh,
    scratch_types=[
        _xbuf, _xbuf, _xbuf,
        _tbuf, _tbuf, _tbuf,
        pltpu.SemaphoreType.DMA,
        pltpu.SemaphoreType.DMA,
        pltpu.SemaphoreType.DMA,
        pltpu.SemaphoreType.DMA,
        pltpu.SemaphoreType.DMA,
        pltpu.SemaphoreType.DMA,
        pltpu.SemaphoreType.DMA,
        pltpu.SemaphoreType.DMA,
        pltpu.SemaphoreType.DMA,
    ],
    compiler_params=pltpu.CompilerParams(use_tc_tiling_on_sc=True),
)
def _sc_add(x_hbm, t_hbm, o_hbm,
            x0, x1, x2, t0, t1, t2,
            lx0, lx1, lx2, lt0, lt1, lt2, ss0, ss1, ss2):
    wid = lax.axis_index("s") * 2 + lax.axis_index("c")
    base = wid * SEQ_PER_W
    xbufs = (x0, x1, x2)
    tbufs = (t0, t1, t2)
    xsems = (lx0, lx1, lx2)
    tsems = (lt0, lt1, lt2)
    ssems = (ss0, ss1, ss2)

    def start_loads(g):
        s = g % NS
        xd = pltpu.async_copy(x_hbm.at[:, pl.ds(base + g * R, R)],
                              xbufs[s], xsems[s])
        td = pltpu.async_copy(t_hbm.at[pl.ds(base + g * R, R)],
                              tbufs[s], tsems[s])
        return xd, td

    loads = [start_loads(0), None, None]
    stores = [None, None, None]

    for g in range(GROUPS):
        s = g % NS
        ng = g + PF
        if ng < GROUPS:
            if stores[ng % NS] is not None:
                stores[ng % NS].wait()
            loads[ng % NS] = start_loads(ng)
        xd, td = loads[s]
        xd.wait()
        td.wait()
        xb = xbufs[s]
        tb = tbufs[s]

        @plsc.parallel_loop(0, R * LANES, unroll=16)
        def _(i):
            r = i // LANES
            sl = pl.ds((i % LANES) * 16, 16)
            t = tb[r, sl]
            for b in range(B):
                plsc.addupdate(xb.at[b, r, sl], t)

        stores[s] = pltpu.async_copy(xb, o_hbm.at[:, pl.ds(base + g * R, R)],
                                     ssems[s])

    for st in stores:
        if st is not None:
            st.wait()


def kernel(x, table):
    return _sc_add(x, table)


# speedup vs baseline: 1.0459x; 1.0441x over previous
import functools
import jax
import jax.numpy as jnp
from jax import lax
from jax.experimental import pallas as pl
from jax.experimental.pallas import tpu as pltpu, tpu_sc as plsc

B, S, D = 4, 8192, 1024
NW = 32
SEQ_PER_W = S // NW
R = 8
GROUPS = SEQ_PER_W // R
LANES = D // 16
NS = 3
PF = 2

_mesh = plsc.VectorSubcoreMesh(core_axis_name="c", subcore_axis_name="s",
                               num_cores=2, num_subcores=16)

_xbuf = pltpu.VMEM((B, R, D), jnp.float32)
_tbuf = pltpu.VMEM((R, D), jnp.float32)


@functools.partial(
    pl.kernel,
    out_type=jax.ShapeDtypeStruct((B, S, D), jnp.float32),
    mesh=_mesh,
    scratch_types=[
        _xbuf, _xbuf, _xbuf,
        _tbuf, _tbuf, _tbuf,
        pltpu.SemaphoreType.DMA,
        pltpu.SemaphoreType.DMA,
        pltpu.SemaphoreType.DMA,
        pltpu.SemaphoreType.DMA,
        pltpu.SemaphoreType.DMA,
        pltpu.SemaphoreType.DMA,
        pltpu.SemaphoreType.DMA,
        pltpu.SemaphoreType.DMA,
        pltpu.SemaphoreType.DMA,
    ],
    compiler_params=pltpu.CompilerParams(use_tc_tiling_on_sc=True),
)
def _sc_add(x_hbm, t_hbm, o_hbm,
            x0, x1, x2, t0, t1, t2,
            lx0, lx1, lx2, lt0, lt1, lt2, ss0, ss1, ss2):
    wid = lax.axis_index("s") * 2 + lax.axis_index("c")
    base = wid * SEQ_PER_W
    xbufs = (x0, x1, x2)
    tbufs = (t0, t1, t2)
    xsems = (lx0, lx1, lx2)
    tsems = (lt0, lt1, lt2)
    ssems = (ss0, ss1, ss2)

    def start_loads(g):
        s = g % NS
        xd = pltpu.async_copy(x_hbm.at[:, pl.ds(base + g * R, R)],
                              xbufs[s], xsems[s])
        td = pltpu.async_copy(t_hbm.at[pl.ds(base + g * R, R)],
                              tbufs[s], tsems[s])
        return xd, td

    loads = [start_loads(0), start_loads(1), None]
    stores = [None, None, None]

    for g in range(GROUPS):
        s = g % NS
        ng = g + PF
        if ng < GROUPS:
            if stores[ng % NS] is not None:
                stores[ng % NS].wait()
            loads[ng % NS] = start_loads(ng)
        xd, td = loads[s]
        xd.wait()
        td.wait()
        xb = xbufs[s]
        tb = tbufs[s]

        @plsc.parallel_loop(0, R * LANES, unroll=8)
        def _(i):
            r = i // LANES
            sl = pl.ds((i % LANES) * 16, 16)
            t = tb[r, sl]
            for b in range(B):
                plsc.addupdate(xb.at[b, r, sl], t)

        stores[s] = pltpu.async_copy(xb, o_hbm.at[:, pl.ds(base + g * R, R)],
                                     ssems[s])

    for st in stores:
        if st is not None:
            st.wait()


def kernel(x, table):
    return _sc_add(x, table)
